# baseline (speedup 1.0000x reference)
import jax
import jax.numpy as jnp
from jax import lax
from jax.experimental import pallas as pl
from jax.experimental.pallas import tpu as pltpu

N = 1024
D = 128
M = 4
TJ = 256


def _gat_kernel(adj_ref, feat_ref, w1_ref, b1_ref, wg_ref, al_ref, ar_ref,
                bg_ref, out_ref, h_ref, el_ref, er_ref):
    oj = pl.program_id(0)
    ij = pl.program_id(1)
    j = oj * (pl.num_programs(1)) + ij

    @pl.when(ij == 0)
    def _feat():
        h0 = jnp.tanh(
            jnp.dot(feat_ref[...], w1_ref[...],
                    precision=lax.Precision.HIGHEST) + b1_ref[...])
        h = jnp.dot(h0, wg_ref[...], precision=lax.Precision.HIGHEST)
        h_ref[...] = h
        el_ref[...] = jnp.dot(h, al_ref[...],
                              precision=lax.Precision.HIGHEST)
        er_ref[...] = lax.dot_general(
            ar_ref[...], h, (((1,), (1,)), ((), ())),
            precision=lax.Precision.HIGHEST)

    msum = (adj_ref[0] + adj_ref[1]) + (adj_ref[2] + adj_ref[3])
    mask = msum != 0.0

    er_tile = er_ref[:, pl.ds(j * TJ, TJ)]
    s = el_ref[...] + er_tile
    e = jnp.maximum(s, 0.2 * s)

    rows = lax.broadcasted_iota(jnp.int32, (N, TJ), 0)
    cols = lax.broadcasted_iota(jnp.int32, (N, TJ), 1) + j * TJ
    diag = rows == cols
    valid = mask | diag

    em = jnp.where(valid, e, -jnp.inf)
    emax = jnp.max(em, axis=0, keepdims=True)
    w = mask.astype(jnp.float32) + diag.astype(jnp.float32)
    ee = jnp.exp(em - emax) * w
    denom = jnp.sum(ee, axis=0, keepdims=True)
    alpha = ee * (1.0 / denom)

    out = lax.dot_general(
        alpha, h_ref[...], (((0,), (0,)), ((), ())),
        precision=lax.Precision.DEFAULT)
    out_ref[...] = jnp.tanh(out + bg_ref[...])


@jax.jit
def kernel(adj_list, feat, attention_weights, W1, b1, Wg, attn_l, attn_r,
           bias_g):
    del attention_weights
    inner = (N // TJ) // 2
    out = pl.pallas_call(
        _gat_kernel,
        grid=(2, inner),
        in_specs=[
            pl.BlockSpec((M, N, TJ), lambda oj, ij: (0, 0, oj * inner + ij)),
            pl.BlockSpec((N, D), lambda oj, ij: (0, 0)),
            pl.BlockSpec((D, D), lambda oj, ij: (0, 0)),
            pl.BlockSpec((1, D), lambda oj, ij: (0, 0)),
            pl.BlockSpec((D, D), lambda oj, ij: (0, 0)),
            pl.BlockSpec((D, 1), lambda oj, ij: (0, 0)),
            pl.BlockSpec((1, D), lambda oj, ij: (0, 0)),
            pl.BlockSpec((1, D), lambda oj, ij: (0, 0)),
        ],
        out_specs=pl.BlockSpec((TJ, D), lambda oj, ij: (oj * inner + ij, 0)),
        out_shape=jax.ShapeDtypeStruct((N, D), jnp.float32),
        scratch_shapes=[
            pltpu.VMEM((N, D), jnp.float32),
            pltpu.VMEM((N, 1), jnp.float32),
            pltpu.VMEM((1, N), jnp.float32),
        ],
        compiler_params=pltpu.CompilerParams(
            dimension_semantics=("parallel", "arbitrary")),
    )(adj_list, feat, W1, b1.reshape(1, D), Wg, attn_l.reshape(D, 1),
      attn_r.reshape(1, D), bias_g.reshape(1, D))
    return out


# speedup vs baseline: 1.1758x; 1.1758x over previous
import jax
import jax.numpy as jnp
from jax import lax
from jax.experimental import pallas as pl
from jax.experimental.pallas import tpu as pltpu

N = 1024
D = 128
M = 4
TJ = 256


def _gat_kernel(adj0_ref, adj1_ref, adj2_ref, adj3_ref, feat_ref, w1_ref,
                b1_ref, wg_ref, al_ref, ar_ref, bg_ref, out_ref, h_ref,
                el_ref, er_ref):
    j = pl.program_id(0)

    @pl.when(j == 0)
    def _feat():
        h0 = jnp.tanh(
            jnp.dot(feat_ref[...], w1_ref[...],
                    precision=lax.Precision.HIGHEST) + b1_ref[...])
        h = jnp.dot(h0, wg_ref[...], precision=lax.Precision.HIGHEST)
        h_ref[...] = h
        el_ref[...] = jnp.dot(h, al_ref[...],
                              precision=lax.Precision.HIGHEST)
        er_ref[...] = lax.dot_general(
            ar_ref[...], h, (((1,), (1,)), ((), ())),
            precision=lax.Precision.HIGHEST)

    msum = (adj0_ref[0] + adj1_ref[0]) + (adj2_ref[0] + adj3_ref[0])
    mask = msum != 0.0

    er_tile = er_ref[:, pl.ds(j * TJ, TJ)]
    s = el_ref[...] + er_tile
    e = jnp.maximum(s, 0.2 * s)

    rows = lax.broadcasted_iota(jnp.int32, (N, TJ), 0)
    cols = lax.broadcasted_iota(jnp.int32, (N, TJ), 1) + j * TJ
    diag = rows == cols
    valid = mask | diag

    em = jnp.where(valid, e, -jnp.inf)
    emax = jnp.max(em, axis=0, keepdims=True)
    w = mask.astype(jnp.float32) + diag.astype(jnp.float32)
    ee = jnp.exp(em - emax) * w
    denom = jnp.sum(ee, axis=0, keepdims=True)
    alpha = ee * (1.0 / denom)

    out = lax.dot_general(
        alpha, h_ref[...], (((0,), (0,)), ((), ())),
        precision=lax.Precision.DEFAULT)
    out_ref[...] = jnp.tanh(out + bg_ref[...])


@jax.jit
def kernel(adj_list, feat, attention_weights, W1, b1, Wg, attn_l, attn_r,
           bias_g):
    del attention_weights
    out = pl.pallas_call(
        _gat_kernel,
        grid=(N // TJ,),
        in_specs=[
            pl.BlockSpec((1, N, TJ), lambda j: (0, 0, j)),
            pl.BlockSpec((1, N, TJ), lambda j: (1, 0, j)),
            pl.BlockSpec((1, N, TJ), lambda j: (2, 0, j)),
            pl.BlockSpec((1, N, TJ), lambda j: (3, 0, j)),
            pl.BlockSpec((N, D), lambda j: (0, 0)),
            pl.BlockSpec((D, D), lambda j: (0, 0)),
            pl.BlockSpec((1, D), lambda j: (0, 0)),
            pl.BlockSpec((D, D), lambda j: (0, 0)),
            pl.BlockSpec((D, 1), lambda j: (0, 0)),
            pl.BlockSpec((1, D), lambda j: (0, 0)),
            pl.BlockSpec((1, D), lambda j: (0, 0)),
        ],
        out_specs=pl.BlockSpec((TJ, D), lambda j: (j, 0)),
        out_shape=jax.ShapeDtypeStruct((N, D), jnp.float32),
        scratch_shapes=[
            pltpu.VMEM((N, D), jnp.float32),
            pltpu.VMEM((N, 1), jnp.float32),
            pltpu.VMEM((1, N), jnp.float32),
        ],
    )(adj_list, adj_list, adj_list, adj_list, feat, W1, b1.reshape(1, D),
      Wg, attn_l.reshape(D, 1), attn_r.reshape(1, D), bias_g.reshape(1, D))
    return out


# speedup vs baseline: 1.4388x; 1.2237x over previous
import jax
import jax.numpy as jnp
from jax import lax
from jax.experimental import pallas as pl
from jax.experimental.pallas import tpu as pltpu

N = 1024
D = 128
M = 4
TJ = 256
NT = N // TJ


def _gat_kernel(adj_hbm, feat_ref, w1_ref, b1_ref, wg_ref, al_ref, ar_ref,
                bg_ref, out_ref, buf_ref, sem, h_ref, el_ref, er_ref):
    def tile_copy(t, slot):
        return pltpu.make_async_copy(
            adj_hbm.at[:, :, pl.ds(t * TJ, TJ)], buf_ref.at[slot],
            sem.at[slot])

    tile_copy(0, 0).start()

    h0 = jnp.tanh(jnp.dot(feat_ref[...], w1_ref[...]) + b1_ref[...])
    h = jnp.dot(h0, wg_ref[...])
    h_ref[...] = h
    el_ref[...] = jnp.dot(h, al_ref[...])
    er_ref[...] = lax.dot_general(
        ar_ref[...], h, (((1,), (1,)), ((), ())))

    for t in range(NT):
        if t + 1 < NT:
            tile_copy(t + 1, (t + 1) % 2).start()
        tile_copy(t, t % 2).wait()
        adj = buf_ref[t % 2]

        msum = (adj[0] + adj[1]) + (adj[2] + adj[3])
        mask = msum != 0.0

        s = el_ref[...] + er_ref[:, t * TJ:(t + 1) * TJ]
        e = jnp.maximum(s, 0.2 * s)

        rows = lax.broadcasted_iota(jnp.int32, (N, TJ), 0)
        cols = lax.broadcasted_iota(jnp.int32, (N, TJ), 1) + t * TJ
        diag = rows == cols
        valid = mask | diag

        em = jnp.where(valid, e, -jnp.inf)
        emax = jnp.max(em, axis=0, keepdims=True)
        w = mask.astype(jnp.float32) + diag.astype(jnp.float32)
        ee = jnp.exp(em - emax) * w
        denom = jnp.sum(ee, axis=0, keepdims=True)
        alpha = ee * (1.0 / denom)

        out = lax.dot_general(alpha, h, (((0,), (0,)), ((), ())))
        out_ref[t * TJ:(t + 1) * TJ, :] = jnp.tanh(out + bg_ref[...])


@jax.jit
def kernel(adj_list, feat, attention_weights, W1, b1, Wg, attn_l, attn_r,
           bias_g):
    del attention_weights
    out = pl.pallas_call(
        _gat_kernel,
        in_specs=[
            pl.BlockSpec(memory_space=pltpu.MemorySpace.HBM),
            pl.BlockSpec((N, D), lambda: (0, 0)),
            pl.BlockSpec((D, D), lambda: (0, 0)),
            pl.BlockSpec((1, D), lambda: (0, 0)),
            pl.BlockSpec((D, D), lambda: (0, 0)),
            pl.BlockSpec((D, 1), lambda: (0, 0)),
            pl.BlockSpec((1, D), lambda: (0, 0)),
            pl.BlockSpec((1, D), lambda: (0, 0)),
        ],
        out_specs=pl.BlockSpec((N, D), lambda: (0, 0)),
        out_shape=jax.ShapeDtypeStruct((N, D), jnp.float32),
        scratch_shapes=[
            pltpu.VMEM((2, M, N, TJ), jnp.float32),
            pltpu.SemaphoreType.DMA((2,)),
            pltpu.VMEM((N, D), jnp.float32),
            pltpu.VMEM((N, 1), jnp.float32),
            pltpu.VMEM((1, N), jnp.float32),
        ],
    )(adj_list, feat, W1, b1.reshape(1, D), Wg, attn_l.reshape(D, 1),
      attn_r.reshape(1, D), bias_g.reshape(1, D))
    return out
